# baseline (speedup 1.0000x reference)
import jax
import jax.numpy as jnp
from jax.experimental import pallas as pl

K = 64
ROW_BLOCK = 64


def _count_ge(vals_s16, cand_s16):
    m = (vals_s16 >= cand_s16)
    c16 = jnp.sum(m.astype(jnp.int16), axis=1, keepdims=True)
    c32 = c16.astype(jnp.int32)
    return jnp.where(c32 == -32768, 32768, c32)


def _topk_mask_kernel(x_ref, o_ref):
    x = x_ref[...]
    R = x.shape[0]
    b = jax.lax.bitcast_convert_type(x, jnp.uint32)
    key = jnp.where(b >= jnp.uint32(0x80000000), ~b, b | jnp.uint32(0x80000000))
    hi = (key >> 16).astype(jnp.int32)
    lo = (key & jnp.uint32(0xFFFF)).astype(jnp.int32)
    h_s = (hi - 32768).astype(jnp.int16)

    thrh = jnp.zeros((R, 1), jnp.int32)
    for bit in range(15, -1, -1):
        cand = thrh | (1 << bit)
        cnt = _count_ge(h_s, (cand - 32768).astype(jnp.int16))
        thrh = jnp.where(cnt >= K, cand, thrh)

    cand_up = jnp.minimum(thrh + 1, 65535)
    cnt_above = _count_ge(h_s, (cand_up - 32768).astype(jnp.int16))
    cnt_above = jnp.where(thrh >= 65535, 0, cnt_above)
    r_needed = K - cnt_above

    p_s = (thrh - 32768).astype(jnp.int16)
    l_m = jnp.where(h_s == p_s, (lo - 32768).astype(jnp.int16),
                    jnp.int16(-32768))
    thrl = jnp.zeros((R, 1), jnp.int32)
    for bit in range(15, -1, -1):
        cand = thrl | (1 << bit)
        cnt = _count_ge(l_m, (cand - 32768).astype(jnp.int16))
        thrl = jnp.where(cnt >= r_needed, cand, thrl)

    thr_key = ((thrh.astype(jnp.uint32) << 16)
               | thrl.astype(jnp.uint32))
    o_ref[...] = jnp.where(key >= thr_key, x, jnp.float32(0.0))


def kernel(x):
    B, N = x.shape
    return pl.pallas_call(
        _topk_mask_kernel,
        grid=(B // ROW_BLOCK,),
        in_specs=[pl.BlockSpec((ROW_BLOCK, N), lambda i: (i, 0))],
        out_specs=pl.BlockSpec((ROW_BLOCK, N), lambda i: (i, 0)),
        out_shape=jax.ShapeDtypeStruct((B, N), x.dtype),
    )(x)


# speedup vs baseline: 2.0913x; 2.0913x over previous
import functools

import jax
import jax.numpy as jnp
import numpy as np
from jax import lax
from jax.experimental import pallas as pl
from jax.experimental.pallas import tpu as pltpu
from jax.experimental.pallas import tpu_sc as plsc

K = 64
ROW_BLOCK = 64
N_COLS = 32768
SC_ROWS = 512
NW = 32
LANES = 16
NV = N_COLS // LANES
UNROLL = 8

I32MIN = np.int32(-2147483648)


def _topk_mask_kernel(x_ref, o_ref):
    x = x_ref[...]
    b = jax.lax.bitcast_convert_type(x, jnp.uint32)
    key = jnp.where(b >= jnp.uint32(0x80000000), ~b, b | jnp.uint32(0x80000000))
    thr = jnp.zeros((x.shape[0], 1), jnp.uint32)
    for bit in range(31, -1, -1):
        cand = thr | jnp.uint32(1 << bit)
        cnt = jnp.sum((key >= cand).astype(jnp.int32), axis=1, keepdims=True)
        thr = jnp.where(cnt >= K, cand, thr)
    o_ref[...] = jnp.where(key >= thr, x, jnp.float32(0.0))


def _tc_topk(x):
    B, N = x.shape
    return pl.pallas_call(
        _topk_mask_kernel,
        grid=(B // ROW_BLOCK,),
        in_specs=[pl.BlockSpec((ROW_BLOCK, N), lambda i: (i, 0))],
        out_specs=pl.BlockSpec((ROW_BLOCK, N), lambda i: (i, 0)),
        out_shape=jax.ShapeDtypeStruct((B, N), x.dtype),
    )(x)


def _gather16(v, idx):
    dn = lax.GatherDimensionNumbers(
        offset_dims=(), collapsed_slice_dims=(0,), start_index_map=(0,))
    return lax.gather(v, idx[:, None], dn, (1,),
                      mode=lax.GatherScatterMode.PROMISE_IN_BOUNDS)


def _butterfly_sum(v):
    for s in (8, 4, 2, 1):
        v = v + _gather16(v, lax.iota(jnp.int32, LANES) ^ s)
    return v


def _f32_key(v):
    b = jax.lax.bitcast_convert_type(v, jnp.int32)
    return jnp.where(b < 0, b ^ jnp.int32(0x7FFFFFFF), b)


def _sc_topk(x):
    rows_pw = SC_ROWS // NW
    mesh = plsc.VectorSubcoreMesh(core_axis_name="c", subcore_axis_name="s")

    @functools.partial(
        pl.kernel,
        mesh=mesh,
        out_type=jax.ShapeDtypeStruct((SC_ROWS, N_COLS), jnp.float32),
        scratch_types=[
            pltpu.VMEM((N_COLS,), jnp.float32),
            pltpu.VMEM((N_COLS,), jnp.int32),
        ],
    )
    def body(x_hbm, out_hbm, row_v, key_v):
        wid = lax.axis_index("s") * 2 + lax.axis_index("c")

        def do_row(i, carry):
            r = wid * rows_pw + i
            pltpu.sync_copy(x_hbm.at[r], row_v)

            def enc_body(j, c):
                base = j * (LANES * UNROLL)
                for u in range(UNROLL):
                    sl = pl.ds(base + u * LANES, LANES)
                    key_v[sl] = _f32_key(row_v[sl])
                return c

            lax.fori_loop(0, NV // UNROLL, enc_body, jnp.int32(0))

            thr = jnp.full((LANES,), I32MIN)
            for bit in range(31, -1, -1):
                step = jnp.int32(I32MIN) if bit == 31 else jnp.int32(1 << bit)
                cand = thr + step

                def cnt_body(j, acc):
                    base = j * (LANES * UNROLL)
                    for u in range(UNROLL):
                        kv = key_v[pl.ds(base + u * LANES, LANES)]
                        acc = acc + jnp.where(kv >= cand, 1, 0)
                    return acc

                acc = lax.fori_loop(0, NV // UNROLL, cnt_body,
                                    jnp.zeros((LANES,), jnp.int32))
                cnt = _butterfly_sum(acc)
                thr = jnp.where(cnt >= K, cand, thr)

            def mask_body(j, c):
                base = j * (LANES * UNROLL)
                for u in range(UNROLL):
                    sl = pl.ds(base + u * LANES, LANES)
                    row_v[sl] = jnp.where(key_v[sl] >= thr, row_v[sl],
                                          jnp.float32(0.0))
                return c

            lax.fori_loop(0, NV // UNROLL, mask_body, jnp.int32(0))
            pltpu.sync_copy(row_v, out_hbm.at[r])
            return carry

        lax.fori_loop(0, rows_pw, do_row, jnp.int32(0))

    return body(x)


def kernel(x):
    B, N = x.shape
    tc_rows = B - SC_ROWS
    out_tc = _tc_topk(x[:tc_rows])
    out_sc = _sc_topk(x[tc_rows:])
    return jnp.concatenate([out_tc, out_sc], axis=0)


# speedup vs baseline: 2.0918x; 1.0002x over previous
import functools

import jax
import jax.numpy as jnp
import numpy as np
from jax import lax
from jax.experimental import pallas as pl
from jax.experimental.pallas import tpu as pltpu
from jax.experimental.pallas import tpu_sc as plsc

K = 64
ROW_BLOCK = 64
N_COLS = 32768
SC_ROWS = 512
NW = 32
LANES = 16
NV = N_COLS // LANES
UNROLL = 8

I32MIN = np.int32(-2147483648)


def _topk_mask_kernel(x_ref, o_ref):
    x = x_ref[...]
    b = jax.lax.bitcast_convert_type(x, jnp.uint32)
    key = jnp.where(b >= jnp.uint32(0x80000000), ~b, b | jnp.uint32(0x80000000))
    thr = jnp.zeros((x.shape[0], 1), jnp.uint32)
    for bit in range(31, -1, -1):
        cand = thr | jnp.uint32(1 << bit)
        cnt = jnp.sum((key >= cand).astype(jnp.int32), axis=1, keepdims=True)
        thr = jnp.where(cnt >= K, cand, thr)
    o_ref[...] = jnp.where(key >= thr, x, jnp.float32(0.0))


def _tc_topk(x):
    B, N = x.shape
    return pl.pallas_call(
        _topk_mask_kernel,
        grid=(B // ROW_BLOCK,),
        in_specs=[pl.BlockSpec((ROW_BLOCK, N), lambda i: (i, 0))],
        out_specs=pl.BlockSpec((ROW_BLOCK, N), lambda i: (i, 0)),
        out_shape=jax.ShapeDtypeStruct((B, N), x.dtype),
    )(x)


def _gather16(v, idx):
    dn = lax.GatherDimensionNumbers(
        offset_dims=(), collapsed_slice_dims=(0,), start_index_map=(0,))
    return lax.gather(v, idx[:, None], dn, (1,),
                      mode=lax.GatherScatterMode.PROMISE_IN_BOUNDS)


def _butterfly_sum(v):
    for s in (8, 4, 2, 1):
        v = v + _gather16(v, lax.iota(jnp.int32, LANES) ^ s)
    return v


def _f32_key(v):
    b = jax.lax.bitcast_convert_type(v, jnp.int32)
    return jnp.where(b < 0, b ^ jnp.int32(0x7FFFFFFF), b)


def _sc_topk(x):
    rows_pw = SC_ROWS // NW
    mesh = plsc.VectorSubcoreMesh(core_axis_name="c", subcore_axis_name="s")

    @functools.partial(
        pl.kernel,
        mesh=mesh,
        out_type=jax.ShapeDtypeStruct((SC_ROWS, N_COLS), jnp.float32),
        scratch_types=[
            pltpu.VMEM((N_COLS,), jnp.float32),
            pltpu.VMEM((N_COLS,), jnp.int32),
        ],
    )
    def body(x_hbm, out_hbm, row_v, key_v):
        wid = lax.axis_index("s") * 2 + lax.axis_index("c")

        def do_row(i, carry):
            r = wid * rows_pw + i
            pltpu.sync_copy(x_hbm.at[r], row_v)

            def enc_body(j, c):
                base = j * (LANES * UNROLL)
                for u in range(UNROLL):
                    sl = pl.ds(base + u * LANES, LANES)
                    key_v[sl] = _f32_key(row_v[sl])
                return c

            lax.fori_loop(0, NV // UNROLL, enc_body, jnp.int32(0))

            thr = jnp.full((LANES,), I32MIN)
            for bit in range(31, -1, -1):
                step = jnp.int32(I32MIN) if bit == 31 else jnp.int32(1 << bit)
                cand = thr + step

                def cnt_body(j, acc):
                    base = j * (LANES * UNROLL)
                    for u in range(UNROLL):
                        kv = key_v[pl.ds(base + u * LANES, LANES)]
                        acc = acc + jnp.where(kv >= cand, 1, 0)
                    return acc

                acc = lax.fori_loop(0, NV // UNROLL, cnt_body,
                                    jnp.zeros((LANES,), jnp.int32))
                cnt = _butterfly_sum(acc)
                thr = jnp.where(cnt >= K, cand, thr)

            def mask_body(j, c):
                base = j * (LANES * UNROLL)
                for u in range(UNROLL):
                    sl = pl.ds(base + u * LANES, LANES)
                    row_v[sl] = jnp.where(key_v[sl] >= thr, row_v[sl],
                                          jnp.float32(0.0))
                return c

            lax.fori_loop(0, NV // UNROLL, mask_body, jnp.int32(0))
            pltpu.sync_copy(row_v, out_hbm.at[r])
            return carry

        lax.fori_loop(0, rows_pw, do_row, jnp.int32(0))

    return body(x)


def kernel(x):
    B, N = x.shape
    tc_rows = B - SC_ROWS
    out_sc = _sc_topk(x[tc_rows:])
    out_tc = _tc_topk(x[:tc_rows])
    return jnp.concatenate([out_tc, out_sc], axis=0)


# speedup vs baseline: 2.5183x; 1.2039x over previous
import jax
import jax.numpy as jnp
from jax.experimental import pallas as pl

K = 64
ROW_BLOCK = 64
CHAINS = 2


def _topk_mask_kernel(x_ref, o_ref):
    x = x_ref[...]
    R = x.shape[0]
    b = jax.lax.bitcast_convert_type(x, jnp.uint32)
    key = jnp.where(b >= jnp.uint32(0x80000000), ~b, b | jnp.uint32(0x80000000))
    rs = R // CHAINS
    keys = [key[c * rs:(c + 1) * rs] for c in range(CHAINS)]
    thrs = [jnp.zeros((rs, 1), jnp.uint32) for _ in range(CHAINS)]
    for bit in range(31, -1, -1):
        for c in range(CHAINS):
            cand = thrs[c] | jnp.uint32(1 << bit)
            cnt = jnp.sum((keys[c] >= cand).astype(jnp.int32), axis=1,
                          keepdims=True)
            thrs[c] = jnp.where(cnt >= K, cand, thrs[c])
    thr = jnp.concatenate(thrs, axis=0)
    o_ref[...] = jnp.where(key >= thr, x, jnp.float32(0.0))


def kernel(x):
    B, N = x.shape
    return pl.pallas_call(
        _topk_mask_kernel,
        grid=(B // ROW_BLOCK,),
        in_specs=[pl.BlockSpec((ROW_BLOCK, N), lambda i: (i, 0))],
        out_specs=pl.BlockSpec((ROW_BLOCK, N), lambda i: (i, 0)),
        out_shape=jax.ShapeDtypeStruct((B, N), x.dtype),
    )(x)
